# Initial kernel scaffold; baseline (speedup 1.0000x reference)
import jax
import jax.numpy as jnp
from jax.experimental import pallas as pl


def kernel(img_flat, position_embedding):
    raise NotImplementedError("write your pallas kernel here")


# speedup vs baseline: 2.1734x; 2.1734x over previous
import functools

import jax
import jax.numpy as jnp
from jax import lax
from jax.experimental import pallas as pl
from jax.experimental.pallas import tpu as pltpu
from jax.experimental.pallas import tpu_sc as plsc

_NC = 2
_NS = 16
_NW = _NC * _NS


@functools.lru_cache(maxsize=None)
def _make_gather(B, D, C):
    b_per_w = B // _NW
    n_chunks = b_per_w // C
    mesh = plsc.VectorSubcoreMesh(core_axis_name="c", subcore_axis_name="s")

    @functools.partial(
        pl.kernel,
        mesh=mesh,
        out_type=jax.ShapeDtypeStruct((B, D), jnp.float32),
        scratch_types=[
            pltpu.VMEM((b_per_w,), jnp.int32),
            pltpu.VMEM((C, D), jnp.float32),
            pltpu.VMEM((C, D), jnp.float32),
            pltpu.SemaphoreType.DMA,
            pltpu.SemaphoreType.DMA,
        ],
    )
    def k(table_hbm, idx_hbm, out_hbm, idx_v, rows0, rows1, sem0, sem1):
        wid = lax.axis_index("s") * _NC + lax.axis_index("c")
        base = wid * b_per_w
        pltpu.sync_copy(idx_hbm.at[pl.ds(base, b_per_w)], idx_v)

        def body(g, _):
            off = g * C
            pltpu.async_copy(
                table_hbm.at[idx_v.at[pl.ds(off, C)]], rows0, sem0
            ).wait()
            pltpu.sync_copy(rows0, out_hbm.at[pl.ds(base + off, C)])
            return _

        lax.fori_loop(0, n_chunks, body, 0)

    return k


def kernel(img_flat, position_embedding):
    batch, seq = img_flat.shape
    d = position_embedding.shape[1]
    idx = img_flat.reshape(-1).astype(jnp.int32)
    out = _make_gather(batch * seq, d, 64)(position_embedding, idx)
    return out.reshape(batch, seq, d)


# speedup vs baseline: 2.3349x; 1.0743x over previous
import functools

import jax
import jax.numpy as jnp
from jax import lax
from jax.experimental import pallas as pl
from jax.experimental.pallas import tpu as pltpu
from jax.experimental.pallas import tpu_sc as plsc

_NC = 2
_NS = 16
_NW = _NC * _NS


@functools.lru_cache(maxsize=None)
def _make_gather(B, D, C):
    b_per_w = B // _NW
    n_chunks = b_per_w // C
    mesh = plsc.VectorSubcoreMesh(core_axis_name="c", subcore_axis_name="s")

    @functools.partial(
        pl.kernel,
        mesh=mesh,
        out_type=jax.ShapeDtypeStruct((B, D), jnp.float32),
        scratch_types=[
            pltpu.VMEM((b_per_w,), jnp.int32),
            pltpu.VMEM((C, D), jnp.float32),
            pltpu.VMEM((C, D), jnp.float32),
            pltpu.SemaphoreType.DMA,
            pltpu.SemaphoreType.DMA,
        ],
    )
    def k(table_hbm, idx_hbm, out_hbm, idx_v, rows0, rows1, sem0, sem1):
        wid = lax.axis_index("s") * _NC + lax.axis_index("c")
        base = wid * b_per_w
        pltpu.sync_copy(idx_hbm.at[pl.ds(base, b_per_w)], idx_v)

        rows = (rows0, rows1)
        sems = (sem0, sem1)

        def gather(g, b):
            pltpu.async_copy(
                table_hbm.at[idx_v.at[pl.ds(g * C, C)]], rows[b], sems[b]
            )

        def drain(b):
            pltpu.make_async_copy(
                table_hbm.at[idx_v.at[pl.ds(0, C)]], rows[b], sems[b]
            ).wait()

        gather(0, 0)
        gather(1, 1)

        def body(i, carry):
            for b in range(2):
                g = i * 2 + b
                drain(b)
                pltpu.sync_copy(rows[b], out_hbm.at[pl.ds(base + g * C, C)])

                @pl.when(g + 2 < n_chunks)
                def _refill(b=b, g=g):
                    gather(g + 2, b)

            return carry

        lax.fori_loop(0, n_chunks // 2, body, 0)

    return k


def kernel(img_flat, position_embedding):
    batch, seq = img_flat.shape
    d = position_embedding.shape[1]
    idx = img_flat.reshape(-1).astype(jnp.int32)
    out = _make_gather(batch * seq, d, 64)(position_embedding, idx)
    return out.reshape(batch, seq, d)
